# baseline (speedup 1.0000x reference)
import functools

import jax
import jax.numpy as jnp
from jax import lax
from jax.experimental import pallas as pl
from jax.experimental.pallas import tpu as pltpu
from jax.experimental.pallas import tpu_sc as plsc

_B = 8
_N = 10000
_E = 640000
_NP = 16384
_ROWS = 8
_COLS = 2048
_NC = 2
_NS = 16
_EPT = _E // 4
_CHUNK = 2000
_NCHUNK = _EPT // _CHUNK
_VPC = _CHUNK // 16

_VMIN = 0.95
_VMAX = 1.05
_PBW = 10.0

_S1, _S2, _S3 = -1.0 / 6.0, 1.0 / 120.0, -1.0 / 5040.0
_C1, _C2, _C3, _C4 = -0.5, 1.0 / 24.0, -1.0 / 720.0, 1.0 / 40320.0


def _pik_body(vm_hbm, va_hbm, r_hbm, x_hbm, pb_hbm, qb_hbm, ij_hbm,
              z_hbm, out_hbm,
              vm_ref, va_ref, acc_p, acc_q,
              ij_buf0, r_buf0, x_buf0,
              ij_buf1, r_buf1, x_buf1,
              rows_a, rows_b, rows_c, rows_d, pb_buf, qb_buf, out_buf,
              shared, sem0, sem1):
    core = lax.axis_index("c")
    sub = lax.axis_index("s")
    wid = core * _NS + sub
    bl = sub // 4
    b = core * 4 + bl
    ec = lax.rem(sub, 4)

    ebase_edge = ec * _EPT
    ebase_rx = b * _E + ec * _EPT

    slot_bufs = ((ij_buf0, r_buf0, x_buf0),
                 (ij_buf1, r_buf1, x_buf1))

    def _edge_copies(ci, slot, sem):
        eo = ebase_edge + ci * _CHUNK
        ro = ebase_rx + ci * _CHUNK
        ijb, rb, xb = slot_bufs[slot]
        return (
            pltpu.make_async_copy(ij_hbm.at[pl.ds(eo, _CHUNK)], ijb, sem),
            pltpu.make_async_copy(r_hbm.at[pl.ds(ro, _CHUNK)], rb, sem),
            pltpu.make_async_copy(x_hbm.at[pl.ds(ro, _CHUNK)], xb, sem),
        )

    def _start(ci, slot, sem):
        for c in _edge_copies(ci, slot, sem):
            c.start()

    def _wait(ci, slot, sem):
        for c in _edge_copies(ci, slot, sem):
            c.wait()

    _start(0, 0, sem0)
    _start(1, 1, sem1)

    pltpu.sync_copy(z_hbm, acc_p)
    pltpu.sync_copy(z_hbm, acc_q)
    pltpu.sync_copy(vm_hbm.at[pl.ds(b * _NP, _NP)], vm_ref)
    pltpu.sync_copy(va_hbm.at[pl.ds(b * _NP, _NP)], va_ref)

    def _compute(slot):
        ijb, rb, xb = slot_bufs[slot]

        @plsc.parallel_loop(0, _CHUNK, step=16, unroll=25)
        def vec_body(o):
            ij = ijb[pl.ds(o, 16)]
            ii = lax.bitwise_and(ij, 0xFFFF)
            jj = lax.shift_right_logical(ij, 16)
            vi = plsc.load_gather(vm_ref, [ii])
            vj = plsc.load_gather(vm_ref, [jj])
            ai = plsc.load_gather(va_ref, [ii])
            aj = plsc.load_gather(va_ref, [jj])
            r = rb[pl.ds(o, 16)]
            x = xb[pl.ds(o, 16)]
            inv = 1.0 / (r * r + x * x)
            th = ai - aj
            t = th * th
            st = th * (1.0 + t * (_S1 + t * (_S2 + t * _S3)))
            ct = 1.0 + t * (_C1 + t * (_C2 + t * (_C3 + t * _C4)))
            w = vi * vj * inv
            p = w * (r * ct - x * st)
            q = w * (r * st + x * ct)
            row = lax.shift_right_logical(ii, 11)
            col = lax.bitwise_and(ii, 2047)
            plsc.addupdate_scatter(acc_p, [row, col], p)
            plsc.addupdate_scatter(acc_q, [row, col], q)

    def pair_body(ci2, carry):
        ci = 2 * ci2
        _wait(ci, 0, sem0)
        _compute(0)

        @pl.when(ci + 2 < _NCHUNK)
        def _():
            _start(ci + 2, 0, sem0)

        _wait(ci + 1, 1, sem1)
        _compute(1)

        @pl.when(ci + 3 < _NCHUNK)
        def _():
            _start(ci + 3, 1, sem1)

        return carry

    lax.fori_loop(0, _NCHUNK // 2, pair_body, 0)

    r0 = lax.rem(2 * sub, 8)
    boff = b * _NP + r0 * _COLS
    pltpu.sync_copy(pb_hbm.at[pl.ds(boff, 2 * _COLS)], pb_buf)
    pltpu.sync_copy(qb_hbm.at[pl.ds(boff, 2 * _COLS)], qb_buf)

    def make_row_body(h, bias_buf):
        def row_body(k, acc):
            o = k * 16
            pc = (rows_a[h, pl.ds(o, 16)] + rows_b[h, pl.ds(o, 16)]
                  + rows_c[h, pl.ds(o, 16)] + rows_d[h, pl.ds(o, 16)])
            bv = bias_buf[pl.ds(h * _COLS + o, 16)]
            d = pc - bv
            return acc + d * d
        return row_body

    pacc = jnp.zeros((16,), jnp.float32)
    for acc_ref, bias_buf in ((acc_p, pb_buf), (acc_q, qb_buf)):
        plsc.subcore_barrier()
        pltpu.sync_copy(acc_ref, shared.at[sub])
        plsc.subcore_barrier()
        pltpu.sync_copy(shared.at[4 * bl, pl.ds(r0, 2)], rows_a)
        pltpu.sync_copy(shared.at[4 * bl + 1, pl.ds(r0, 2)], rows_b)
        pltpu.sync_copy(shared.at[4 * bl + 2, pl.ds(r0, 2)], rows_c)
        pltpu.sync_copy(shared.at[4 * bl + 3, pl.ds(r0, 2)], rows_d)
        pacc = lax.fori_loop(0, _COLS // 16, make_row_body(0, bias_buf), pacc)
        pacc = lax.fori_loop(0, _COLS // 16, make_row_body(1, bias_buf), pacc)

    def volt_body(k, acc):
        v = vm_ref[pl.ds(ec * (_NP // 4) + k * 16, 16)]
        lo = jnp.maximum(_VMIN - v, 0.0)
        hi = jnp.maximum(v - _VMAX, 0.0)
        return acc + lo * lo + hi * hi

    vacc = lax.fori_loop(0, _NP // 4 // 16, volt_body,
                         jnp.zeros((16,), jnp.float32))

    out_buf[...] = (_PBW * pacc + vacc) * (1.0 / _B)
    pltpu.sync_copy(out_buf, out_hbm.at[pl.ds(wid * 16, 16)])


_pik = functools.partial(
    pl.kernel,
    out_type=jax.ShapeDtypeStruct((_NC * _NS * 16,), jnp.float32),
    mesh=plsc.VectorSubcoreMesh(core_axis_name="c", subcore_axis_name="s"),
    compiler_params=pltpu.CompilerParams(needs_layout_passes=False),
    scratch_types=[
        pltpu.VMEM((_NP,), jnp.float32),
        pltpu.VMEM((_NP,), jnp.float32),
        pltpu.VMEM((_ROWS, _COLS), jnp.float32),
        pltpu.VMEM((_ROWS, _COLS), jnp.float32),
        pltpu.VMEM((_CHUNK,), jnp.int32),
        pltpu.VMEM((_CHUNK,), jnp.float32),
        pltpu.VMEM((_CHUNK,), jnp.float32),
        pltpu.VMEM((_CHUNK,), jnp.int32),
        pltpu.VMEM((_CHUNK,), jnp.float32),
        pltpu.VMEM((_CHUNK,), jnp.float32),
        pltpu.VMEM((2, _COLS), jnp.float32),
        pltpu.VMEM((2, _COLS), jnp.float32),
        pltpu.VMEM((2, _COLS), jnp.float32),
        pltpu.VMEM((2, _COLS), jnp.float32),
        pltpu.VMEM((2 * _COLS,), jnp.float32),
        pltpu.VMEM((2 * _COLS,), jnp.float32),
        pltpu.VMEM((16,), jnp.float32),
        pltpu.VMEM_SHARED((_NS, _ROWS, _COLS), jnp.float32),
        pltpu.SemaphoreType.DMA,
        pltpu.SemaphoreType.DMA,
    ],
)(_pik_body)


def kernel(v_mag, v_ang, r_line, x_line, p_bus, q_bus, edge_index):
    pad = ((0, 0), (0, _NP - _N))
    vm = jnp.pad(v_mag, pad, constant_values=1.0).reshape(-1)
    va = jnp.pad(v_ang, pad).reshape(-1)
    pb = jnp.pad(p_bus, pad).reshape(-1)
    qb = jnp.pad(q_bus, pad).reshape(-1)
    r = r_line.reshape(-1)
    x = x_line.reshape(-1)
    ij = jnp.bitwise_or(edge_index[0],
                        jnp.left_shift(edge_index[1], 16))
    zeros = jnp.zeros((_ROWS, _COLS), jnp.float32)
    partials = _pik(vm, va, r, x, pb, qb, ij, zeros)
    loss = jnp.sum(partials)
    return (v_mag, v_ang, loss)


# speedup vs baseline: 1.2021x; 1.2021x over previous
import functools

import jax
import jax.numpy as jnp
from jax import lax
from jax.experimental import pallas as pl
from jax.experimental.pallas import tpu as pltpu
from jax.experimental.pallas import tpu_sc as plsc

_B = 8
_N = 10000
_E = 640000
_NC = 2
_NS = 16
_EPT = _E // 4
_CHUNK = 2000
_NCHUNK = _EPT // _CHUNK

_NODE_CHUNKS = ((0, 2512), (2512, 2512), (5024, 2512), (7536, 2464))
_MAXL = 2512

_VMIN = 0.95
_VMAX = 1.05

_S1, _S2, _S3 = -1.0 / 6.0, 1.0 / 120.0, -1.0 / 5040.0
_C1, _C2, _C3, _C4 = -0.5, 1.0 / 24.0, -1.0 / 720.0, 1.0 / 40320.0


def _pik_body(vm_hbm, va_hbm, r_hbm, x_hbm, pb_hbm, qb_hbm, e_hbm,
              out_hbm,
              vm_ref, va_ref, acc_p, acc_q,
              i_buf0, j_buf0, r_buf0, x_buf0,
              i_buf1, j_buf1, r_buf1, x_buf1,
              rows_a, rows_b, rows_c, rows_d, bias_buf, accvec,
              shared, sem0, sem1):
    core = lax.axis_index("c")
    sub = lax.axis_index("s")
    wid = core * _NS + sub
    bl = sub // 4
    b = core * 4 + bl
    ec = lax.rem(sub, 4)

    ebase = ec * _EPT
    ebase_rx = b * _E + ec * _EPT

    slot_bufs = ((i_buf0, j_buf0, r_buf0, x_buf0),
                 (i_buf1, j_buf1, r_buf1, x_buf1))

    def _edge_copies(ci, slot, sem):
        eo = ebase + ci * _CHUNK
        ro = ebase_rx + ci * _CHUNK
        ib, jb, rb, xb = slot_bufs[slot]
        return (
            pltpu.make_async_copy(e_hbm.at[pl.ds(eo, _CHUNK)], ib, sem),
            pltpu.make_async_copy(e_hbm.at[pl.ds(_E + eo, _CHUNK)], jb, sem),
            pltpu.make_async_copy(r_hbm.at[pl.ds(ro, _CHUNK)], rb, sem),
            pltpu.make_async_copy(x_hbm.at[pl.ds(ro, _CHUNK)], xb, sem),
        )

    def _start(ci, slot, sem):
        for c in _edge_copies(ci, slot, sem):
            c.start()

    def _wait(ci, slot, sem):
        for c in _edge_copies(ci, slot, sem):
            c.wait()

    _start(0, 0, sem0)
    _start(1, 1, sem1)

    pltpu.sync_copy(vm_hbm.at[pl.ds(b * _N, _N)], vm_ref)
    pltpu.sync_copy(va_hbm.at[pl.ds(b * _N, _N)], va_ref)

    @plsc.parallel_loop(0, _N, step=16)
    def _zero(o):
        z = jnp.zeros((16,), jnp.float32)
        acc_p[pl.ds(o, 16)] = z
        acc_q[pl.ds(o, 16)] = z

    def _compute(slot):
        ib, jb, rb, xb = slot_bufs[slot]

        @plsc.parallel_loop(0, _CHUNK, step=16, unroll=5)
        def vec_body(o):
            ii = ib[pl.ds(o, 16)]
            jj = jb[pl.ds(o, 16)]
            vi = plsc.load_gather(vm_ref, [ii])
            vj = plsc.load_gather(vm_ref, [jj])
            ai = plsc.load_gather(va_ref, [ii])
            aj = plsc.load_gather(va_ref, [jj])
            r = rb[pl.ds(o, 16)]
            x = xb[pl.ds(o, 16)]
            inv = 1.0 / (r * r + x * x)
            th = ai - aj
            t = th * th
            st = th * (1.0 + t * (_S1 + t * (_S2 + t * _S3)))
            ct = 1.0 + t * (_C1 + t * (_C2 + t * (_C3 + t * _C4)))
            w = vi * vj * inv
            p = w * (r * ct - x * st)
            q = w * (r * st + x * ct)
            plsc.addupdate_scatter(acc_p, [ii], p)
            plsc.addupdate_scatter(acc_q, [ii], q)

    def pair_body(ci2, carry):
        ci = 2 * ci2
        _wait(ci, 0, sem0)
        _compute(0)

        @pl.when(ci + 2 < _NCHUNK)
        def _():
            _start(ci + 2, 0, sem0)

        _wait(ci + 1, 1, sem1)
        _compute(1)

        @pl.when(ci + 3 < _NCHUNK)
        def _():
            _start(ci + 3, 1, sem1)

        return carry

    lax.fori_loop(0, _NCHUNK // 2, pair_body, 0)

    accvec[...] = jnp.zeros((16,), jnp.float32)

    for acc_ref, bias_hbm in ((acc_p, pb_hbm), (acc_q, qb_hbm)):
        plsc.subcore_barrier()
        pltpu.sync_copy(acc_ref, shared.at[pl.ds(sub * _N, _N)])
        plsc.subcore_barrier()

        for v, (o, ln) in enumerate(_NODE_CHUNKS):
            @pl.when(ec == v)
            def _(o=o, ln=ln):
                base = 4 * bl * _N + o
                pltpu.sync_copy(shared.at[pl.ds(base, ln)],
                                rows_a.at[pl.ds(0, ln)])
                pltpu.sync_copy(shared.at[pl.ds(base + _N, ln)],
                                rows_b.at[pl.ds(0, ln)])
                pltpu.sync_copy(shared.at[pl.ds(base + 2 * _N, ln)],
                                rows_c.at[pl.ds(0, ln)])
                pltpu.sync_copy(shared.at[pl.ds(base + 3 * _N, ln)],
                                rows_d.at[pl.ds(0, ln)])
                pltpu.sync_copy(bias_hbm.at[pl.ds(b * _N + o, ln)],
                                bias_buf.at[pl.ds(0, ln)])

                def mis_body(k, a):
                    kk = k * 16
                    pc = (rows_a[pl.ds(kk, 16)] + rows_b[pl.ds(kk, 16)]
                          + rows_c[pl.ds(kk, 16)] + rows_d[pl.ds(kk, 16)])
                    d = pc - bias_buf[pl.ds(kk, 16)]
                    return a + d * d

                part = lax.fori_loop(0, ln // 16, mis_body,
                                     jnp.zeros((16,), jnp.float32))
                accvec[...] += part

    for v, (o, ln) in enumerate(_NODE_CHUNKS):
        @pl.when(ec == v)
        def _(o=o, ln=ln):
            def volt_body(k, a):
                vmv = vm_ref[pl.ds(o + k * 16, 16)]
                lo = jnp.maximum(_VMIN - vmv, 0.0)
                hi = jnp.maximum(vmv - _VMAX, 0.0)
                return a + lo * lo + hi * hi

            part = lax.fori_loop(0, ln // 16, volt_body,
                                 jnp.zeros((16,), jnp.float32))
            accvec[...] += 0.1 * part

    accvec[...] = accvec[...] * 1.25
    pltpu.sync_copy(accvec, out_hbm.at[pl.ds(wid * 16, 16)])


_pik = functools.partial(
    pl.kernel,
    out_type=jax.ShapeDtypeStruct((_NC * _NS * 16,), jnp.float32),
    mesh=plsc.VectorSubcoreMesh(core_axis_name="c", subcore_axis_name="s"),
    compiler_params=pltpu.CompilerParams(needs_layout_passes=False),
    scratch_types=[
        pltpu.VMEM((_N,), jnp.float32),
        pltpu.VMEM((_N,), jnp.float32),
        pltpu.VMEM((_N,), jnp.float32),
        pltpu.VMEM((_N,), jnp.float32),
        pltpu.VMEM((_CHUNK,), jnp.int32),
        pltpu.VMEM((_CHUNK,), jnp.int32),
        pltpu.VMEM((_CHUNK,), jnp.float32),
        pltpu.VMEM((_CHUNK,), jnp.float32),
        pltpu.VMEM((_CHUNK,), jnp.int32),
        pltpu.VMEM((_CHUNK,), jnp.int32),
        pltpu.VMEM((_CHUNK,), jnp.float32),
        pltpu.VMEM((_CHUNK,), jnp.float32),
        pltpu.VMEM((_MAXL,), jnp.float32),
        pltpu.VMEM((_MAXL,), jnp.float32),
        pltpu.VMEM((_MAXL,), jnp.float32),
        pltpu.VMEM((_MAXL,), jnp.float32),
        pltpu.VMEM((_MAXL,), jnp.float32),
        pltpu.VMEM((16,), jnp.float32),
        pltpu.VMEM_SHARED((_NS * _N,), jnp.float32),
        pltpu.SemaphoreType.DMA,
        pltpu.SemaphoreType.DMA,
    ],
)(_pik_body)


def kernel(v_mag, v_ang, r_line, x_line, p_bus, q_bus, edge_index):
    partials = _pik(v_mag.reshape(-1), v_ang.reshape(-1),
                    r_line.reshape(-1), x_line.reshape(-1),
                    p_bus.reshape(-1), q_bus.reshape(-1),
                    edge_index.reshape(-1))
    loss = jnp.sum(partials)
    return (v_mag, v_ang, loss)


# speedup vs baseline: 1.2145x; 1.0103x over previous
import functools

import jax
import jax.numpy as jnp
from jax import lax
from jax.experimental import pallas as pl
from jax.experimental.pallas import tpu as pltpu
from jax.experimental.pallas import tpu_sc as plsc

_B = 8
_N = 10000
_E = 640000
_NC = 2
_NS = 16
_EPT = _E // 4
_CHUNK = 4000
_NCHUNK = _EPT // _CHUNK

_NODE_CHUNKS = ((0, 2512), (2512, 2512), (5024, 2512), (7536, 2464))
_MAXL = 2512

_VMIN = 0.95
_VMAX = 1.05

_S1, _S2, _S3 = -1.0 / 6.0, 1.0 / 120.0, -1.0 / 5040.0
_C1, _C2, _C3, _C4 = -0.5, 1.0 / 24.0, -1.0 / 720.0, 1.0 / 40320.0


def _pik_body(vm_hbm, va_hbm, r_hbm, x_hbm, pb_hbm, qb_hbm, e_hbm,
              out_hbm,
              vm_ref, va_ref, acc_p, acc_q,
              i_buf0, j_buf0, r_buf0, x_buf0,
              i_buf1, j_buf1, r_buf1, x_buf1,
              rows_a, rows_b, rows_c, rows_d, bias_buf, accvec,
              shared, sem0, sem1):
    core = lax.axis_index("c")
    sub = lax.axis_index("s")
    wid = core * _NS + sub
    bl = sub // 4
    b = core * 4 + bl
    ec = lax.rem(sub, 4)

    ebase = ec * _EPT
    ebase_rx = b * _E + ec * _EPT

    slot_bufs = ((i_buf0, j_buf0, r_buf0, x_buf0),
                 (i_buf1, j_buf1, r_buf1, x_buf1))

    def _edge_copies(ci, slot, sem):
        eo = ebase + ci * _CHUNK
        ro = ebase_rx + ci * _CHUNK
        ib, jb, rb, xb = slot_bufs[slot]
        return (
            pltpu.make_async_copy(e_hbm.at[pl.ds(eo, _CHUNK)], ib, sem),
            pltpu.make_async_copy(e_hbm.at[pl.ds(_E + eo, _CHUNK)], jb, sem),
            pltpu.make_async_copy(r_hbm.at[pl.ds(ro, _CHUNK)], rb, sem),
            pltpu.make_async_copy(x_hbm.at[pl.ds(ro, _CHUNK)], xb, sem),
        )

    def _start(ci, slot, sem):
        for c in _edge_copies(ci, slot, sem):
            c.start()

    def _wait(ci, slot, sem):
        for c in _edge_copies(ci, slot, sem):
            c.wait()

    _start(0, 0, sem0)
    _start(1, 1, sem1)

    pltpu.sync_copy(vm_hbm.at[pl.ds(b * _N, _N)], vm_ref)
    pltpu.sync_copy(va_hbm.at[pl.ds(b * _N, _N)], va_ref)

    @plsc.parallel_loop(0, _N, step=16)
    def _zero(o):
        z = jnp.zeros((16,), jnp.float32)
        acc_p[pl.ds(o, 16)] = z
        acc_q[pl.ds(o, 16)] = z

    def _compute(slot):
        ib, jb, rb, xb = slot_bufs[slot]

        @plsc.parallel_loop(0, _CHUNK, step=16, unroll=5)
        def vec_body(o):
            ii = ib[pl.ds(o, 16)]
            jj = jb[pl.ds(o, 16)]
            vi = plsc.load_gather(vm_ref, [ii])
            vj = plsc.load_gather(vm_ref, [jj])
            ai = plsc.load_gather(va_ref, [ii])
            aj = plsc.load_gather(va_ref, [jj])
            r = rb[pl.ds(o, 16)]
            x = xb[pl.ds(o, 16)]
            inv = 1.0 / (r * r + x * x)
            th = ai - aj
            t = th * th
            st = th * (1.0 + t * (_S1 + t * (_S2 + t * _S3)))
            ct = 1.0 + t * (_C1 + t * (_C2 + t * (_C3 + t * _C4)))
            w = vi * vj * inv
            p = w * (r * ct - x * st)
            q = w * (r * st + x * ct)
            plsc.addupdate_scatter(acc_p, [ii], p)
            plsc.addupdate_scatter(acc_q, [ii], q)

    def pair_body(ci2, carry):
        ci = 2 * ci2
        _wait(ci, 0, sem0)
        _compute(0)

        @pl.when(ci + 2 < _NCHUNK)
        def _():
            _start(ci + 2, 0, sem0)

        _wait(ci + 1, 1, sem1)
        _compute(1)

        @pl.when(ci + 3 < _NCHUNK)
        def _():
            _start(ci + 3, 1, sem1)

        return carry

    lax.fori_loop(0, _NCHUNK // 2, pair_body, 0)

    accvec[...] = jnp.zeros((16,), jnp.float32)

    for acc_ref, bias_hbm in ((acc_p, pb_hbm), (acc_q, qb_hbm)):
        plsc.subcore_barrier()
        pltpu.sync_copy(acc_ref, shared.at[pl.ds(sub * _N, _N)])
        plsc.subcore_barrier()

        for v, (o, ln) in enumerate(_NODE_CHUNKS):
            @pl.when(ec == v)
            def _(o=o, ln=ln):
                base = 4 * bl * _N + o
                pltpu.sync_copy(shared.at[pl.ds(base, ln)],
                                rows_a.at[pl.ds(0, ln)])
                pltpu.sync_copy(shared.at[pl.ds(base + _N, ln)],
                                rows_b.at[pl.ds(0, ln)])
                pltpu.sync_copy(shared.at[pl.ds(base + 2 * _N, ln)],
                                rows_c.at[pl.ds(0, ln)])
                pltpu.sync_copy(shared.at[pl.ds(base + 3 * _N, ln)],
                                rows_d.at[pl.ds(0, ln)])
                pltpu.sync_copy(bias_hbm.at[pl.ds(b * _N + o, ln)],
                                bias_buf.at[pl.ds(0, ln)])

                def mis_body(k, a):
                    kk = k * 16
                    pc = (rows_a[pl.ds(kk, 16)] + rows_b[pl.ds(kk, 16)]
                          + rows_c[pl.ds(kk, 16)] + rows_d[pl.ds(kk, 16)])
                    d = pc - bias_buf[pl.ds(kk, 16)]
                    return a + d * d

                part = lax.fori_loop(0, ln // 16, mis_body,
                                     jnp.zeros((16,), jnp.float32))
                accvec[...] += part

    for v, (o, ln) in enumerate(_NODE_CHUNKS):
        @pl.when(ec == v)
        def _(o=o, ln=ln):
            def volt_body(k, a):
                vmv = vm_ref[pl.ds(o + k * 16, 16)]
                lo = jnp.maximum(_VMIN - vmv, 0.0)
                hi = jnp.maximum(vmv - _VMAX, 0.0)
                return a + lo * lo + hi * hi

            part = lax.fori_loop(0, ln // 16, volt_body,
                                 jnp.zeros((16,), jnp.float32))
            accvec[...] += 0.1 * part

    accvec[...] = accvec[...] * 1.25
    pltpu.sync_copy(accvec, out_hbm.at[pl.ds(wid * 16, 16)])


_pik = functools.partial(
    pl.kernel,
    out_type=jax.ShapeDtypeStruct((_NC * _NS * 16,), jnp.float32),
    mesh=plsc.VectorSubcoreMesh(core_axis_name="c", subcore_axis_name="s"),
    compiler_params=pltpu.CompilerParams(needs_layout_passes=False),
    scratch_types=[
        pltpu.VMEM((_N,), jnp.float32),
        pltpu.VMEM((_N,), jnp.float32),
        pltpu.VMEM((_N,), jnp.float32),
        pltpu.VMEM((_N,), jnp.float32),
        pltpu.VMEM((_CHUNK,), jnp.int32),
        pltpu.VMEM((_CHUNK,), jnp.int32),
        pltpu.VMEM((_CHUNK,), jnp.float32),
        pltpu.VMEM((_CHUNK,), jnp.float32),
        pltpu.VMEM((_CHUNK,), jnp.int32),
        pltpu.VMEM((_CHUNK,), jnp.int32),
        pltpu.VMEM((_CHUNK,), jnp.float32),
        pltpu.VMEM((_CHUNK,), jnp.float32),
        pltpu.VMEM((_MAXL,), jnp.float32),
        pltpu.VMEM((_MAXL,), jnp.float32),
        pltpu.VMEM((_MAXL,), jnp.float32),
        pltpu.VMEM((_MAXL,), jnp.float32),
        pltpu.VMEM((_MAXL,), jnp.float32),
        pltpu.VMEM((16,), jnp.float32),
        pltpu.VMEM_SHARED((_NS * _N,), jnp.float32),
        pltpu.SemaphoreType.DMA,
        pltpu.SemaphoreType.DMA,
    ],
)(_pik_body)


def kernel(v_mag, v_ang, r_line, x_line, p_bus, q_bus, edge_index):
    partials = _pik(v_mag.reshape(-1), v_ang.reshape(-1),
                    r_line.reshape(-1), x_line.reshape(-1),
                    p_bus.reshape(-1), q_bus.reshape(-1),
                    edge_index.reshape(-1))
    loss = jnp.sum(partials)
    return (v_mag, v_ang, loss)
